# baseline (speedup 1.0000x reference)
import functools

import jax
import jax.numpy as jnp
from jax import lax
from jax.experimental import pallas as pl
from jax.experimental.pallas import tpu as pltpu
from jax.experimental.pallas import tpu_sc as plsc

N = 10000
E = 320000
H = 128
EPS = 1e-5

NC, NS = 2, 16
NW = NC * NS
CH = 2
EC = E // CH
PERW = EC // NW
GK = 40
NGK = PERW // GK
NPT = 624
NTAIL = N - NS * NPT
SKI = 40
SNG = PERW // SKI

BE = 1600
GEDGE = EC // BE
NB = 2000
GNODE = N // NB

F32 = jnp.float32
BF16 = jnp.bfloat16


def _mesh():
    return plsc.VectorSubcoreMesh(core_axis_name="c", subcore_axis_name="s",
                                  num_cores=NC, num_subcores=NS)


@functools.partial(
    pl.kernel,
    out_type=(jax.ShapeDtypeStruct((EC, H), F32),
              jax.ShapeDtypeStruct((EC, H), F32)),
    mesh=_mesh(),
    scratch_types=[
        pltpu.VMEM_SHARED((N, H), F32),
        pltpu.VMEM((GK, H), F32),
        pltpu.VMEM((GK, H), F32),
        pltpu.VMEM((PERW,), jnp.int32),
        pltpu.VMEM((PERW,), jnp.int32),
        pltpu.SemaphoreType.DMA,
        pltpu.SemaphoreType.DMA,
        pltpu.SemaphoreType.DMA,
    ],
)
def _sc_gather(table, srcr, dstr, hs_out, hd_out,
               tbl, buf_s, buf_d, idx_s, idx_d, gsem, osem_s, osem_d):
    c = lax.axis_index("c")
    s = lax.axis_index("s")
    w = s * NC + c

    pltpu.sync_copy(table.at[pl.ds(s * NPT, NPT)], tbl.at[pl.ds(s * NPT, NPT)])

    @pl.when(s == NS - 1)
    def _():
        pltpu.sync_copy(table.at[pl.ds(NS * NPT, NTAIL)],
                        tbl.at[pl.ds(NS * NPT, NTAIL)])

    pltpu.sync_copy(srcr.at[w], idx_s)
    pltpu.sync_copy(dstr.at[w], idx_d)
    plsc.subcore_barrier()

    def unit(g, buf, idx, out, osem):
        base = w * PERW + g * GK

        @pl.when(g >= 1)
        def _():
            pltpu.make_async_copy(buf, out.at[pl.ds(base, GK)], osem).wait()

        pltpu.async_copy(tbl.at[idx.at[pl.ds(g * GK, GK)]], buf, gsem).wait()
        pltpu.async_copy(buf, out.at[pl.ds(base, GK)], osem)

    def body(g, carry):
        unit(g, buf_s, idx_s, hs_out, osem_s)
        unit(g, buf_d, idx_d, hd_out, osem_d)
        return carry

    lax.fori_loop(0, NGK, body, 0)
    pltpu.make_async_copy(buf_s, hs_out.at[pl.ds(w * PERW, GK)],
                          osem_s).wait()
    pltpu.make_async_copy(buf_d, hd_out.at[pl.ds(w * PERW, GK)],
                          osem_d).wait()


@functools.partial(
    pl.kernel,
    out_type=jax.ShapeDtypeStruct((NC, N, H), F32),
    mesh=_mesh(),
    scratch_types=[
        pltpu.VMEM_SHARED((N, H), F32),
        pltpu.VMEM((SKI, H), F32),
        pltpu.VMEM((SKI, H), F32),
        pltpu.VMEM((SKI, H), F32),
        pltpu.VMEM((SNG, SKI), jnp.int32),
        pltpu.SemaphoreType.DMA,
        pltpu.SemaphoreType.DMA,
        pltpu.SemaphoreType.DMA,
        pltpu.SemaphoreType.DMA,
        pltpu.SemaphoreType.DMA,
        pltpu.SemaphoreType.DMA,
    ],
)
def _sc_scatter(msg, dstr, zeros, part, acc, rows0, rows1, rows2, idx,
                ssem0, ssem1, ssem2, isem0, isem1, isem2):
    c = lax.axis_index("c")
    s = lax.axis_index("s")
    w = s * NC + c
    rows = (rows0, rows1, rows2)
    ssem = (ssem0, ssem1, ssem2)
    isem = (isem0, isem1, isem2)

    pltpu.sync_copy(zeros.at[pl.ds(s * NPT, NPT)], acc.at[pl.ds(s * NPT, NPT)])

    @pl.when(s == NS - 1)
    def _():
        pltpu.sync_copy(zeros.at[pl.ds(NS * NPT, NTAIL)],
                        acc.at[pl.ds(NS * NPT, NTAIL)])

    pltpu.sync_copy(dstr.at[w], idx)
    plsc.subcore_barrier()

    def _when(cond, fn):
        if isinstance(cond, bool):
            if cond:
                fn()
        else:
            pl.when(cond)(fn)

    def fill(g, k):
        pltpu.async_copy(msg.at[pl.ds(w * PERW + g * SKI, SKI)], rows[k],
                         isem[k])

    def step(g, k):
        pltpu.make_async_copy(msg.at[pl.ds(w * PERW, SKI)], rows[k],
                              isem[k]).wait()
        pltpu.async_copy(rows[k], acc.at[idx.at[g]], ssem[k], add=True)
        _when(g >= 1, lambda: pltpu.make_async_copy(
            rows[(k + 2) % 3], acc.at[idx.at[g]], ssem[(k + 2) % 3]).wait())
        _when(g + 2 < SNG, lambda: fill(g + 2, (k + 2) % 3))

    fill(0, 0)
    fill(1, 1)

    def body(gg, carry):
        step(3 * gg, 0)
        step(3 * gg + 1, 1)
        step(3 * gg + 2, 2)
        return carry

    lax.fori_loop(0, SNG // 3, body, 0)
    step(SNG - 2, 0)
    step(SNG - 1, 1)
    pltpu.make_async_copy(rows[1], acc.at[idx.at[0]], ssem[1]).wait()
    plsc.subcore_barrier()
    pltpu.sync_copy(acc.at[pl.ds(s * NPT, NPT)],
                    part.at[c, pl.ds(s * NPT, NPT)])

    @pl.when(s == NS - 1)
    def _():
        pltpu.sync_copy(acc.at[pl.ds(NS * NPT, NTAIL)],
                        part.at[c, pl.ds(NS * NPT, NTAIL)])


def _dot(a, b):
    return jnp.dot(a, b, preferred_element_type=F32)


def _relu(v):
    return jnp.maximum(v, 0.0)


def _rep(shape):
    return pl.BlockSpec(shape, lambda i: tuple(0 for _ in shape))


def _blk(shape):
    return pl.BlockSpec(shape, lambda i: (i,) + tuple(0 for _ in shape[1:]))


def _tc_params():
    return pltpu.CompilerParams(dimension_semantics=("arbitrary",))


def _proj_body(x_ref, w_ref, b_ref, o_ref, o16_ref):
    v = _relu(_dot(x_ref[...].astype(BF16), w_ref[...]) + b_ref[...])
    o_ref[...] = v
    o16_ref[...] = v.astype(BF16)


def _proj(x, w, b):
    return pl.pallas_call(
        _proj_body,
        grid=(GNODE,),
        in_specs=[_blk((NB, H)), _rep((H, H)), _rep((1, H))],
        out_specs=[_blk((NB, H)), _blk((NB, H))],
        out_shape=[jax.ShapeDtypeStruct((N, H), F32),
                   jax.ShapeDtypeStruct((N, H), BF16)],
        compiler_params=_tc_params(),
    )(x.astype(F32), w.astype(BF16), b)


def _edge_layer(mode, e_srcs, wts):
    def body(*refs):
        if mode == 0:
            (ea_ref, hs_ref, hd_ref,
             we, be, w1s, w1d, w1e, b1, w2, b2,
             v1d, v1s, v1e, c1, v2, c2, v3, c3,
             d_ref, m_ref, s1_ref, s2_ref) = refs
            e_in = _relu(_dot(ea_ref[...].astype(BF16), we[...]) + be[...])
        elif mode == 1:
            (d0_ref, hs_ref, hd_ref, sc0, sh0,
             w1s, w1d, w1e, b1, w2, b2,
             v1d, v1s, v1e, c1, v2, c2, v3, c3,
             d_ref, m_ref, s1_ref, s2_ref) = refs
            e_in = _relu(d0_ref[...].astype(F32) * sc0[...] + sh0[...])
        else:
            (d0_ref, d1_ref, hs_ref, hd_ref, sc0, sh0, sc1, sh1,
             w1s, w1d, w1e, b1, w2, b2,
             v1d, v1s, v1e, c1, v2, c2, v3, c3,
             d_ref, m_ref, s1_ref, s2_ref) = refs
            e_in = (_relu(d0_ref[...].astype(F32) * sc0[...] + sh0[...])
                    + _relu(d1_ref[...].astype(F32) * sc1[...] + sh1[...]))
        e16 = e_in.astype(BF16)
        hsv = hs_ref[...].astype(BF16)
        hdv = hd_ref[...].astype(BF16)
        t = _relu(_dot(hsv, w1s[...]) + _dot(hdv, w1d[...])
                  + _dot(e16, w1e[...]) + b1[...])
        d16 = (_dot(t.astype(BF16), w2[...]) + b2[...]).astype(BF16)
        d_ref[...] = d16
        d = d16.astype(F32)
        m = _relu(_dot(hdv, v1d[...]) + _dot(hsv, v1s[...])
                  + _dot(d16, v1e[...]) + c1[...])
        m = _relu(_dot(m.astype(BF16), v2[...]) + c2[...])
        m_ref[...] = _dot(m.astype(BF16), v3[...]) + c3[...]

        @pl.when(pl.program_id(0) == 0)
        def _():
            s1_ref[...] = jnp.zeros_like(s1_ref)
            s2_ref[...] = jnp.zeros_like(s2_ref)

        s1_ref[...] += jnp.sum(d, axis=0, keepdims=True)
        s2_ref[...] += jnp.sum(d * d, axis=0, keepdims=True)

    esrc_specs = {0: [_blk((BE, 16)), _blk((BE, H)), _blk((BE, H))],
                  1: [_blk((BE, H))] * 3 + [_rep((1, H))] * 2,
                  2: [_blk((BE, H))] * 4 + [_rep((1, H))] * 4}[mode]
    w_specs = [_rep(w.shape) for w in wts]
    out_shapes = [jax.ShapeDtypeStruct((EC, H), BF16),
                  jax.ShapeDtypeStruct((EC, H), F32),
                  jax.ShapeDtypeStruct((1, H), F32),
                  jax.ShapeDtypeStruct((1, H), F32)]
    out_specs = [_blk((BE, H)), _blk((BE, H)), _rep((1, H)), _rep((1, H))]
    return pl.pallas_call(
        body,
        grid=(GEDGE,),
        in_specs=esrc_specs + w_specs,
        out_specs=out_specs,
        out_shape=out_shapes,
        compiler_params=_tc_params(),
    )(*e_srcs, *wts)


def _node_layer(h, part_a, part_b, w1h, w1a, b1, w2, b2, g, bb, residual):
    def body(h_ref, p_ref, q_ref, w1h_r, w1a_r, b1_r, w2_r, b2_r, g_r, bb_r,
             o_ref, o16_ref):
        agg = p_ref[0] + p_ref[1] + q_ref[0] + q_ref[1]
        u = _relu(_dot(h_ref[...].astype(BF16), w1h_r[...])
                  + _dot(agg.astype(BF16), w1a_r[...]) + b1_r[...])
        r = _dot(u.astype(BF16), w2_r[...]) + b2_r[...]
        mean = jnp.mean(r, axis=0, keepdims=True)
        var = jnp.mean(r * r, axis=0, keepdims=True) - mean * mean
        sc = g_r[...] / jnp.sqrt(var + EPS)
        sh = bb_r[...] - mean * sc
        v = _relu(r * sc + sh)
        if residual:
            v = h_ref[...] + v
        o_ref[...] = v
        o16_ref[...] = v.astype(BF16)

    return pl.pallas_call(
        body,
        grid=(1,),
        in_specs=[_rep((N, H)), _rep((NC, N, H)), _rep((NC, N, H)),
                  _rep((H, H)), _rep((H, H)), _rep((1, H)),
                  _rep((H, H)), _rep((1, H)), _rep((1, H)), _rep((1, H))],
        out_specs=[_rep((N, H)), _rep((N, H))],
        out_shape=[jax.ShapeDtypeStruct((N, H), F32),
                   jax.ShapeDtypeStruct((N, H), BF16)],
        compiler_params=_tc_params(),
    )(h, part_a, part_b, w1h, w1a, b1, w2, b2, g, bb)


def _edge_readout(ds_list, scs, shs, a1, a1b, a2, a2b):
    def body(d0_ref, d1_ref, d2_ref, sc0, sh0, sc1, sh1, sc2, sh2,
             a1_r, a1b_r, a2_r, a2b_r, e_ref, at_ref):
        e3 = (_relu(d0_ref[...].astype(F32) * sc0[...] + sh0[...])
              + _relu(d1_ref[...].astype(F32) * sc1[...] + sh1[...])
              + _relu(d2_ref[...].astype(F32) * sc2[...] + sh2[...]))
        e_ref[...] = e3
        t = _relu(_dot(e3.astype(BF16), a1_r[...]) + a1b_r[...])
        logit = _dot(t.astype(BF16), a2_r[...]) + a2b_r[...]
        at_ref[...] = jax.nn.sigmoid(logit)

    return pl.pallas_call(
        body,
        grid=(GEDGE,),
        in_specs=[_blk((BE, H))] * 3 + [_rep((1, H))] * 6
        + [_rep((H, 64)), _rep((1, 64)), _rep((64, 1)), _rep((1, 1))],
        out_specs=[_blk((BE, H)), _blk((BE, 1))],
        out_shape=[jax.ShapeDtypeStruct((EC, H), F32),
                   jax.ShapeDtypeStruct((EC, 1), F32)],
        compiler_params=_tc_params(),
    )(*ds_list, scs[0], shs[0], scs[1], shs[1], scs[2], shs[2],
      a1, a1b, a2, a2b)


def _node_readout(h3, a1, a1b, a2, a2b, t1, t1b, t2, t2b, wo, bo):
    def body(h_ref, a1_r, a1b_r, a2_r, a2b_r, t1_r, t1b_r, t2_r, t2b_r,
             wo_r, bo_r, ge_ref, at_ref, tt_ref):
        h = h_ref[...]
        la = _dot(_relu(_dot(h, a1_r[...]) + a1b_r[...]), a2_r[...]) + a2b_r[...]
        m = jnp.max(la, axis=0, keepdims=True)
        p = jnp.exp(la - m)
        attn = p / jnp.sum(p, axis=0, keepdims=True)
        at_ref[...] = attn
        lt = _dot(_relu(_dot(h, t1_r[...]) + t1b_r[...]), t2_r[...]) + t2b_r[...]
        tt_ref[...] = jax.nn.sigmoid(lt)
        ge = jnp.sum(h * attn, axis=0, keepdims=True)
        ge_ref[...] = _dot(ge, wo_r[...]) + bo_r[...]

    return pl.pallas_call(
        body,
        grid=(1,),
        in_specs=[_rep((N, H)),
                  _rep((H, 64)), _rep((1, 64)), _rep((64, 1)), _rep((1, 1)),
                  _rep((H, 64)), _rep((1, 64)), _rep((64, 1)), _rep((1, 1)),
                  _rep((H, H)), _rep((1, H))],
        out_specs=[_rep((1, H)), _rep((N, 1)), _rep((N, 1))],
        out_shape=[jax.ShapeDtypeStruct((1, H), F32),
                   jax.ShapeDtypeStruct((N, 1), F32),
                   jax.ShapeDtypeStruct((N, 1), F32)],
        compiler_params=_tc_params(),
    )(h3, a1, a1b, a2, a2b, t1, t1b, t2, t2b, wo, bo)


def _row(v):
    return v.reshape(1, -1)


def _bn_scale_shift(s1, s2, g, b):
    mean = s1 / E
    var = s2 / E - mean * mean
    scale = _row(g) / jnp.sqrt(var + EPS)
    shift = _row(b) - mean * scale
    return scale, shift


def kernel(x, edge_index, edge_attr, params):
    src = edge_index[0].astype(jnp.int32)
    dst = edge_index[1].astype(jnp.int32)
    srcr = [src[c * EC:(c + 1) * EC].reshape(NW, PERW) for c in range(CH)]
    dstr = [dst[c * EC:(c + 1) * EC].reshape(NW, PERW) for c in range(CH)]
    dstr_s = [dst[c * EC:(c + 1) * EC].reshape(NW, SNG, SKI)
              for c in range(CH)]
    zeros = jnp.zeros((N, H), F32)

    h, h16 = _proj(x, params["node_in"]["w"], _row(params["node_in"]["b"]))

    def b16(w):
        return w.astype(BF16)

    ds_list, scs, shs = [], [], []
    for i, lp in enumerate(params["layers"]):
        w1 = lp["edge_upd"][0]["w"]
        v1 = lp["edge_mlp"][0]["w"]
        wts = [b16(w1[:H]), b16(w1[H:2 * H]), b16(w1[2 * H:]),
               _row(lp["edge_upd"][0]["b"]),
               b16(lp["edge_upd"][1]["w"]), _row(lp["edge_upd"][1]["b"]),
               b16(v1[:H]), b16(v1[H:2 * H]), b16(v1[2 * H:]),
               _row(lp["edge_mlp"][0]["b"]),
               b16(lp["edge_mlp"][1]["w"]), _row(lp["edge_mlp"][1]["b"]),
               b16(lp["edge_mlp"][2]["w"]), _row(lp["edge_mlp"][2]["b"])]
        if i == 0:
            wts = [b16(params["edge_in"]["w"]),
                   _row(params["edge_in"]["b"])] + wts
        dch, msgch, s1ch, s2ch = [], [], [], []
        gathered = [_sc_gather(h, srcr[c], dstr[c]) for c in range(CH)]
        for c in range(CH):
            hs, hd = gathered[c]
            if i == 0:
                ea_c = edge_attr[c * EC:(c + 1) * EC]
                d, msg, s1, s2 = _edge_layer(0, [ea_c, hs, hd], wts)
            elif i == 1:
                d, msg, s1, s2 = _edge_layer(
                    1, [ds_list[0][c], hs, hd, scs[0], shs[0]], wts)
            else:
                d, msg, s1, s2 = _edge_layer(
                    2, [ds_list[0][c], ds_list[1][c], hs, hd,
                        scs[0], shs[0], scs[1], shs[1]], wts)
            dch.append(d)
            msgch.append(msg)
            s1ch.append(s1)
            s2ch.append(s2)
        ds_list.append(dch)
        e_sc, e_sh = _bn_scale_shift(s1ch[0] + s1ch[1], s2ch[0] + s2ch[1],
                                     lp["bn_edge"]["g"], lp["bn_edge"]["b"])
        scs.append(e_sc)
        shs.append(e_sh)

        parts = [_sc_scatter(msgch[c], dstr_s[c], zeros) for c in range(CH)]
        nw1 = lp["node_mlp"][0]["w"]
        h, h16 = _node_layer(h, parts[0], parts[1],
                             b16(nw1[:H]), b16(nw1[H:]),
                             _row(lp["node_mlp"][0]["b"]),
                             b16(lp["node_mlp"][1]["w"]),
                             _row(lp["node_mlp"][1]["b"]),
                             _row(lp["bn_node"]["g"]),
                             _row(lp["bn_node"]["b"]),
                             residual=(i > 0))

    ro = [_edge_readout(
        [ds_list[0][c], ds_list[1][c], ds_list[2][c]], scs, shs,
        b16(params["edge_attn"][0]["w"]), _row(params["edge_attn"][0]["b"]),
        b16(params["edge_attn"][1]["w"]), _row(params["edge_attn"][1]["b"]))
        for c in range(CH)]
    e3 = jnp.concatenate([ro[0][0], ro[1][0]], axis=0)
    eattn = jnp.concatenate([ro[0][1], ro[1][1]], axis=0)
    ge, nattn, taint = _node_readout(
        h,
        params["node_attn"][0]["w"], _row(params["node_attn"][0]["b"]),
        params["node_attn"][1]["w"], _row(params["node_attn"][1]["b"]),
        params["taint"][0]["w"], _row(params["taint"][0]["b"]),
        params["taint"][1]["w"], _row(params["taint"][1]["b"]),
        params["out_proj"]["w"], _row(params["out_proj"]["b"]))
    return (ge, h, e3, nattn, eattn, taint)


# speedup vs baseline: 1.0076x; 1.0076x over previous
import functools

import jax
import jax.numpy as jnp
from jax import lax
from jax.experimental import pallas as pl
from jax.experimental.pallas import tpu as pltpu
from jax.experimental.pallas import tpu_sc as plsc

N = 10000
E = 320000
H = 128
EPS = 1e-5

NC, NS = 2, 16
NW = NC * NS
PERW = E // NW
GK = 80
NGK = PERW // GK
NPT = 624
NTAIL = N - NS * NPT
SKI = 80
SNG = PERW // SKI

BE = 1600
GEDGE = E // BE
NB = 2000
GNODE = N // NB

F32 = jnp.float32
BF16 = jnp.bfloat16


def _mesh():
    return plsc.VectorSubcoreMesh(core_axis_name="c", subcore_axis_name="s",
                                  num_cores=NC, num_subcores=NS)


@functools.partial(
    pl.kernel,
    out_type=(jax.ShapeDtypeStruct((E, H), F32),
              jax.ShapeDtypeStruct((E, H), F32)),
    mesh=_mesh(),
    scratch_types=[
        pltpu.VMEM_SHARED((N, H), F32),
        pltpu.VMEM((GK, H), F32),
        pltpu.VMEM((GK, H), F32),
        pltpu.VMEM((PERW,), jnp.int32),
        pltpu.VMEM((PERW,), jnp.int32),
        pltpu.SemaphoreType.DMA,
        pltpu.SemaphoreType.DMA,
        pltpu.SemaphoreType.DMA,
    ],
)
def _sc_gather(table, srcr, dstr, hs_out, hd_out,
               tbl, buf_s, buf_d, idx_s, idx_d, gsem, osem_s, osem_d):
    c = lax.axis_index("c")
    s = lax.axis_index("s")
    w = s * NC + c

    pltpu.sync_copy(table.at[pl.ds(s * NPT, NPT)], tbl.at[pl.ds(s * NPT, NPT)])

    @pl.when(s == NS - 1)
    def _():
        pltpu.sync_copy(table.at[pl.ds(NS * NPT, NTAIL)],
                        tbl.at[pl.ds(NS * NPT, NTAIL)])

    pltpu.sync_copy(srcr.at[w], idx_s)
    pltpu.sync_copy(dstr.at[w], idx_d)
    plsc.subcore_barrier()

    def unit(g, buf, idx, out, osem):
        base = w * PERW + g * GK

        @pl.when(g >= 1)
        def _():
            pltpu.make_async_copy(buf, out.at[pl.ds(base, GK)], osem).wait()

        pltpu.async_copy(tbl.at[idx.at[pl.ds(g * GK, GK)]], buf, gsem).wait()
        pltpu.async_copy(buf, out.at[pl.ds(base, GK)], osem)

    def body(g, carry):
        unit(g, buf_s, idx_s, hs_out, osem_s)
        unit(g, buf_d, idx_d, hd_out, osem_d)
        return carry

    lax.fori_loop(0, NGK, body, 0)
    pltpu.make_async_copy(buf_s, hs_out.at[pl.ds(w * PERW, GK)],
                          osem_s).wait()
    pltpu.make_async_copy(buf_d, hd_out.at[pl.ds(w * PERW, GK)],
                          osem_d).wait()


@functools.partial(
    pl.kernel,
    out_type=jax.ShapeDtypeStruct((NC, N, H), F32),
    mesh=_mesh(),
    scratch_types=[
        pltpu.VMEM_SHARED((N, H), F32),
        pltpu.VMEM((SKI, H), F32),
        pltpu.VMEM((SKI, H), F32),
        pltpu.VMEM((SKI, H), F32),
        pltpu.VMEM((SNG, SKI), jnp.int32),
        pltpu.SemaphoreType.DMA,
        pltpu.SemaphoreType.DMA,
        pltpu.SemaphoreType.DMA,
        pltpu.SemaphoreType.DMA,
        pltpu.SemaphoreType.DMA,
        pltpu.SemaphoreType.DMA,
    ],
)
def _sc_scatter(msg, dstr, zeros, part, acc, rows0, rows1, rows2, idx,
                ssem0, ssem1, ssem2, isem0, isem1, isem2):
    c = lax.axis_index("c")
    s = lax.axis_index("s")
    w = s * NC + c
    rows = (rows0, rows1, rows2)
    ssem = (ssem0, ssem1, ssem2)
    isem = (isem0, isem1, isem2)

    pltpu.sync_copy(zeros.at[pl.ds(s * NPT, NPT)], acc.at[pl.ds(s * NPT, NPT)])

    @pl.when(s == NS - 1)
    def _():
        pltpu.sync_copy(zeros.at[pl.ds(NS * NPT, NTAIL)],
                        acc.at[pl.ds(NS * NPT, NTAIL)])

    pltpu.sync_copy(dstr.at[w], idx)
    plsc.subcore_barrier()

    def _when(cond, fn):
        if isinstance(cond, bool):
            if cond:
                fn()
        else:
            pl.when(cond)(fn)

    def fill(g, k):
        pltpu.async_copy(msg.at[pl.ds(w * PERW + g * SKI, SKI)], rows[k],
                         isem[k])

    def step(g, k):
        pltpu.make_async_copy(msg.at[pl.ds(w * PERW, SKI)], rows[k],
                              isem[k]).wait()
        pltpu.async_copy(rows[k], acc.at[idx.at[g]], ssem[k], add=True)
        _when(g >= 1, lambda: pltpu.make_async_copy(
            rows[(k + 2) % 3], acc.at[idx.at[g]], ssem[(k + 2) % 3]).wait())
        _when(g + 2 < SNG, lambda: fill(g + 2, (k + 2) % 3))

    fill(0, 0)
    fill(1, 1)

    def body(gg, carry):
        step(3 * gg, 0)
        step(3 * gg + 1, 1)
        step(3 * gg + 2, 2)
        return carry

    lax.fori_loop(0, SNG // 3, body, 0)
    step(SNG - 2, 0)
    step(SNG - 1, 1)
    pltpu.make_async_copy(rows[1], acc.at[idx.at[0]], ssem[1]).wait()
    plsc.subcore_barrier()
    pltpu.sync_copy(acc.at[pl.ds(s * NPT, NPT)],
                    part.at[c, pl.ds(s * NPT, NPT)])

    @pl.when(s == NS - 1)
    def _():
        pltpu.sync_copy(acc.at[pl.ds(NS * NPT, NTAIL)],
                        part.at[c, pl.ds(NS * NPT, NTAIL)])


def _dot(a, b):
    return jnp.dot(a, b, preferred_element_type=F32)


def _relu(v):
    return jnp.maximum(v, 0.0)


def _rep(shape):
    return pl.BlockSpec(shape, lambda i: tuple(0 for _ in shape))


def _blk(shape):
    return pl.BlockSpec(shape, lambda i: (i,) + tuple(0 for _ in shape[1:]))


def _tc_params():
    return pltpu.CompilerParams(dimension_semantics=("arbitrary",))


def _proj_body(x_ref, w_ref, b_ref, o_ref, o16_ref):
    v = _relu(_dot(x_ref[...].astype(BF16), w_ref[...]) + b_ref[...])
    o_ref[...] = v
    o16_ref[...] = v.astype(BF16)


def _proj(x, w, b):
    return pl.pallas_call(
        _proj_body,
        grid=(GNODE,),
        in_specs=[_blk((NB, H)), _rep((H, H)), _rep((1, H))],
        out_specs=[_blk((NB, H)), _blk((NB, H))],
        out_shape=[jax.ShapeDtypeStruct((N, H), F32),
                   jax.ShapeDtypeStruct((N, H), BF16)],
        compiler_params=_tc_params(),
    )(x.astype(F32), w.astype(BF16), b)


def _edge_layer(mode, e_srcs, wts):
    def body(*refs):
        if mode == 0:
            (ea_ref, hs_ref, hd_ref,
             we, be, w1s, w1d, w1e, b1, w2, b2,
             v1d, v1s, v1e, c1, v2, c2, v3, c3,
             d_ref, m_ref, s1_ref, s2_ref) = refs
            e_in = _relu(_dot(ea_ref[...].astype(BF16), we[...]) + be[...])
        elif mode == 1:
            (d0_ref, hs_ref, hd_ref, sc0, sh0,
             w1s, w1d, w1e, b1, w2, b2,
             v1d, v1s, v1e, c1, v2, c2, v3, c3,
             d_ref, m_ref, s1_ref, s2_ref) = refs
            e_in = _relu(d0_ref[...].astype(F32) * sc0[...] + sh0[...])
        else:
            (d0_ref, d1_ref, hs_ref, hd_ref, sc0, sh0, sc1, sh1,
             w1s, w1d, w1e, b1, w2, b2,
             v1d, v1s, v1e, c1, v2, c2, v3, c3,
             d_ref, m_ref, s1_ref, s2_ref) = refs
            e_in = (_relu(d0_ref[...].astype(F32) * sc0[...] + sh0[...])
                    + _relu(d1_ref[...].astype(F32) * sc1[...] + sh1[...]))
        e16 = e_in.astype(BF16)
        hsv = hs_ref[...].astype(BF16)
        hdv = hd_ref[...].astype(BF16)
        t = _relu(_dot(hsv, w1s[...]) + _dot(hdv, w1d[...])
                  + _dot(e16, w1e[...]) + b1[...])
        d16 = (_dot(t.astype(BF16), w2[...]) + b2[...]).astype(BF16)
        d_ref[...] = d16
        d = d16.astype(F32)
        m = _relu(_dot(hdv, v1d[...]) + _dot(hsv, v1s[...])
                  + _dot(d16, v1e[...]) + c1[...])
        m = _relu(_dot(m.astype(BF16), v2[...]) + c2[...])
        m_ref[...] = _dot(m.astype(BF16), v3[...]) + c3[...]

        @pl.when(pl.program_id(0) == 0)
        def _():
            s1_ref[...] = jnp.zeros_like(s1_ref)
            s2_ref[...] = jnp.zeros_like(s2_ref)

        s1_ref[...] += jnp.sum(d, axis=0, keepdims=True)
        s2_ref[...] += jnp.sum(d * d, axis=0, keepdims=True)

    esrc_specs = {0: [_blk((BE, 16)), _blk((BE, H)), _blk((BE, H))],
                  1: [_blk((BE, H))] * 3 + [_rep((1, H))] * 2,
                  2: [_blk((BE, H))] * 4 + [_rep((1, H))] * 4}[mode]
    w_specs = [_rep(w.shape) for w in wts]
    out_shapes = [jax.ShapeDtypeStruct((E, H), BF16),
                  jax.ShapeDtypeStruct((E, H), F32),
                  jax.ShapeDtypeStruct((1, H), F32),
                  jax.ShapeDtypeStruct((1, H), F32)]
    out_specs = [_blk((BE, H)), _blk((BE, H)), _rep((1, H)), _rep((1, H))]
    return pl.pallas_call(
        body,
        grid=(GEDGE,),
        in_specs=esrc_specs + w_specs,
        out_specs=out_specs,
        out_shape=out_shapes,
        compiler_params=_tc_params(),
    )(*e_srcs, *wts)


def _node_layer(h, part, w1h, w1a, b1, w2, b2, g, bb, residual):
    def body(h_ref, p_ref, w1h_r, w1a_r, b1_r, w2_r, b2_r, g_r, bb_r,
             o_ref, o16_ref):
        agg = p_ref[0] + p_ref[1]
        u = _relu(_dot(h_ref[...].astype(BF16), w1h_r[...])
                  + _dot(agg.astype(BF16), w1a_r[...]) + b1_r[...])
        r = _dot(u.astype(BF16), w2_r[...]) + b2_r[...]
        mean = jnp.mean(r, axis=0, keepdims=True)
        var = jnp.mean(r * r, axis=0, keepdims=True) - mean * mean
        sc = g_r[...] / jnp.sqrt(var + EPS)
        sh = bb_r[...] - mean * sc
        v = _relu(r * sc + sh)
        if residual:
            v = h_ref[...] + v
        o_ref[...] = v
        o16_ref[...] = v.astype(BF16)

    return pl.pallas_call(
        body,
        grid=(1,),
        in_specs=[_rep((N, H)), _rep((NC, N, H)),
                  _rep((H, H)), _rep((H, H)), _rep((1, H)),
                  _rep((H, H)), _rep((1, H)), _rep((1, H)), _rep((1, H))],
        out_specs=[_rep((N, H)), _rep((N, H))],
        out_shape=[jax.ShapeDtypeStruct((N, H), F32),
                   jax.ShapeDtypeStruct((N, H), BF16)],
        compiler_params=_tc_params(),
    )(h, part, w1h, w1a, b1, w2, b2, g, bb)


def _edge_readout(ds_list, scs, shs, a1, a1b, a2, a2b):
    def body(d0_ref, d1_ref, d2_ref, sc0, sh0, sc1, sh1, sc2, sh2,
             a1_r, a1b_r, a2_r, a2b_r, e_ref, at_ref):
        e3 = (_relu(d0_ref[...].astype(F32) * sc0[...] + sh0[...])
              + _relu(d1_ref[...].astype(F32) * sc1[...] + sh1[...])
              + _relu(d2_ref[...].astype(F32) * sc2[...] + sh2[...]))
        e_ref[...] = e3
        t = _relu(_dot(e3.astype(BF16), a1_r[...]) + a1b_r[...])
        logit = _dot(t.astype(BF16), a2_r[...]) + a2b_r[...]
        at_ref[...] = jax.nn.sigmoid(logit)

    return pl.pallas_call(
        body,
        grid=(GEDGE,),
        in_specs=[_blk((BE, H))] * 3 + [_rep((1, H))] * 6
        + [_rep((H, 64)), _rep((1, 64)), _rep((64, 1)), _rep((1, 1))],
        out_specs=[_blk((BE, H)), _blk((BE, 1))],
        out_shape=[jax.ShapeDtypeStruct((E, H), F32),
                   jax.ShapeDtypeStruct((E, 1), F32)],
        compiler_params=_tc_params(),
    )(*ds_list, scs[0], shs[0], scs[1], shs[1], scs[2], shs[2],
      a1, a1b, a2, a2b)


def _node_readout(h3, a1, a1b, a2, a2b, t1, t1b, t2, t2b, wo, bo):
    def body(h_ref, a1_r, a1b_r, a2_r, a2b_r, t1_r, t1b_r, t2_r, t2b_r,
             wo_r, bo_r, ge_ref, at_ref, tt_ref):
        h = h_ref[...]
        la = _dot(_relu(_dot(h, a1_r[...]) + a1b_r[...]), a2_r[...]) + a2b_r[...]
        m = jnp.max(la, axis=0, keepdims=True)
        p = jnp.exp(la - m)
        attn = p / jnp.sum(p, axis=0, keepdims=True)
        at_ref[...] = attn
        lt = _dot(_relu(_dot(h, t1_r[...]) + t1b_r[...]), t2_r[...]) + t2b_r[...]
        tt_ref[...] = jax.nn.sigmoid(lt)
        ge = jnp.sum(h * attn, axis=0, keepdims=True)
        ge_ref[...] = _dot(ge, wo_r[...]) + bo_r[...]

    return pl.pallas_call(
        body,
        grid=(1,),
        in_specs=[_rep((N, H)),
                  _rep((H, 64)), _rep((1, 64)), _rep((64, 1)), _rep((1, 1)),
                  _rep((H, 64)), _rep((1, 64)), _rep((64, 1)), _rep((1, 1)),
                  _rep((H, H)), _rep((1, H))],
        out_specs=[_rep((1, H)), _rep((N, 1)), _rep((N, 1))],
        out_shape=[jax.ShapeDtypeStruct((1, H), F32),
                   jax.ShapeDtypeStruct((N, 1), F32),
                   jax.ShapeDtypeStruct((N, 1), F32)],
        compiler_params=_tc_params(),
    )(h3, a1, a1b, a2, a2b, t1, t1b, t2, t2b, wo, bo)


def _row(v):
    return v.reshape(1, -1)


def _bn_scale_shift(s1, s2, g, b):
    mean = s1 / E
    var = s2 / E - mean * mean
    scale = _row(g) / jnp.sqrt(var + EPS)
    shift = _row(b) - mean * scale
    return scale, shift


def kernel(x, edge_index, edge_attr, params):
    src = edge_index[0].astype(jnp.int32)
    dst = edge_index[1].astype(jnp.int32)
    srcr = src.reshape(NW, PERW)
    dstr = dst.reshape(NW, PERW)
    dstr_s = dst.reshape(NW, SNG, SKI)
    zeros = jnp.zeros((N, H), F32)

    h, h16 = _proj(x, params["node_in"]["w"], _row(params["node_in"]["b"]))

    def b16(w):
        return w.astype(BF16)

    ds_list, scs, shs = [], [], []
    for i, lp in enumerate(params["layers"]):
        hs, hd = _sc_gather(h, srcr, dstr)
        w1 = lp["edge_upd"][0]["w"]
        v1 = lp["edge_mlp"][0]["w"]
        wts = [b16(w1[:H]), b16(w1[H:2 * H]), b16(w1[2 * H:]),
               _row(lp["edge_upd"][0]["b"]),
               b16(lp["edge_upd"][1]["w"]), _row(lp["edge_upd"][1]["b"]),
               b16(v1[:H]), b16(v1[H:2 * H]), b16(v1[2 * H:]),
               _row(lp["edge_mlp"][0]["b"]),
               b16(lp["edge_mlp"][1]["w"]), _row(lp["edge_mlp"][1]["b"]),
               b16(lp["edge_mlp"][2]["w"]), _row(lp["edge_mlp"][2]["b"])]
        if i == 0:
            wts = [b16(params["edge_in"]["w"]),
                   _row(params["edge_in"]["b"])] + wts
            d, msg, s1, s2 = _edge_layer(0, [edge_attr, hs, hd], wts)
        elif i == 1:
            d, msg, s1, s2 = _edge_layer(
                1, [ds_list[0], hs, hd, scs[0], shs[0]], wts)
        else:
            d, msg, s1, s2 = _edge_layer(
                2, [ds_list[0], ds_list[1], hs, hd,
                    scs[0], shs[0], scs[1], shs[1]], wts)
        ds_list.append(d)
        e_sc, e_sh = _bn_scale_shift(s1, s2, lp["bn_edge"]["g"],
                                     lp["bn_edge"]["b"])
        scs.append(e_sc)
        shs.append(e_sh)

        part = _sc_scatter(msg, dstr_s, zeros)
        nw1 = lp["node_mlp"][0]["w"]
        h, h16 = _node_layer(h, part, b16(nw1[:H]), b16(nw1[H:]),
                             _row(lp["node_mlp"][0]["b"]),
                             b16(lp["node_mlp"][1]["w"]),
                             _row(lp["node_mlp"][1]["b"]),
                             _row(lp["bn_node"]["g"]),
                             _row(lp["bn_node"]["b"]),
                             residual=(i > 0))

    e3, eattn = _edge_readout(
        ds_list, scs, shs,
        b16(params["edge_attn"][0]["w"]), _row(params["edge_attn"][0]["b"]),
        b16(params["edge_attn"][1]["w"]), _row(params["edge_attn"][1]["b"]))
    ge, nattn, taint = _node_readout(
        h,
        params["node_attn"][0]["w"], _row(params["node_attn"][0]["b"]),
        params["node_attn"][1]["w"], _row(params["node_attn"][1]["b"]),
        params["taint"][0]["w"], _row(params["taint"][0]["b"]),
        params["taint"][1]["w"], _row(params["taint"][1]["b"]),
        params["out_proj"]["w"], _row(params["out_proj"]["b"]))
    return (ge, h, e3, nattn, eattn, taint)
